# baseline (device time: 17601 ns/iter reference)
import jax
import jax.numpy as jnp
from jax import lax
from jax.experimental import pallas as pl
from jax.experimental.pallas import tpu as pltpu

N_DEV = 16


def _max_hops(d):
    plane = {0: 0, 1: 1, 2: 2, 3: 1}[d & 3]
    return plane + {0: 0, 1: 1, 2: 2, 3: 3}[d >> 2]


_FAR_FIRST = sorted(range(1, N_DEV), key=_max_hops, reverse=True)


def kernel(x):
    m, n = x.shape
    rows = m // N_DEV

    def body(x_ref, out_ref, rs_buf, red_ref,
             rs_send, rs_recv, ag_send, ag_recv):
        my = lax.axis_index("i")

        barrier = pltpu.get_barrier_semaphore()
        for d in _FAR_FIRST:
            pl.semaphore_signal(
                barrier,
                inc=1,
                device_id=(my ^ d,),
                device_id_type=pl.DeviceIdType.MESH,
            )
        pl.semaphore_wait(barrier, N_DEV - 1)

        rs = [None] * N_DEV
        for d in _FAR_FIRST:
            peer = my ^ d
            r = pltpu.make_async_remote_copy(
                src_ref=x_ref.at[pl.ds(peer * rows, rows), :],
                dst_ref=rs_buf.at[d],
                send_sem=rs_send.at[d],
                recv_sem=rs_recv.at[d],
                device_id=(peer,),
                device_id_type=pl.DeviceIdType.MESH,
            )
            r.start()
            rs[d] = r

        rs_buf[0] = x_ref[pl.ds(my * rows, rows), :]
        for d in range(1, N_DEV):
            rs[d].wait_recv()
        rs_buf[0:8] = rs_buf[0:8] + rs_buf[8:16]
        rs_buf[0:4] = rs_buf[0:4] + rs_buf[4:8]
        rs_buf[0:2] = rs_buf[0:2] + rs_buf[2:4]
        red_ref[...] = rs_buf[0] + rs_buf[1]

        ag = []
        for d in _FAR_FIRST:
            peer = my ^ d
            r = pltpu.make_async_remote_copy(
                src_ref=red_ref,
                dst_ref=out_ref.at[pl.ds(my * rows, rows), :],
                send_sem=ag_send.at[d],
                recv_sem=ag_recv.at[d],
                device_id=(peer,),
                device_id_type=pl.DeviceIdType.MESH,
            )
            r.start()
            ag.append(r)

        out_ref[pl.ds(my * rows, rows), :] = red_ref[...]
        for r in ag:
            r.wait_recv()

        for d in range(1, N_DEV):
            rs[d].wait_send()
        for r in ag:
            r.wait_send()

    return pl.pallas_call(
        body,
        out_shape=jax.ShapeDtypeStruct((m, n), x.dtype),
        in_specs=[pl.BlockSpec(memory_space=pltpu.VMEM)],
        out_specs=pl.BlockSpec(memory_space=pltpu.VMEM),
        scratch_shapes=[
            pltpu.VMEM((N_DEV, rows, n), x.dtype),
            pltpu.VMEM((rows, n), x.dtype),
            pltpu.SemaphoreType.DMA((N_DEV,)),
            pltpu.SemaphoreType.DMA((N_DEV,)),
            pltpu.SemaphoreType.DMA((N_DEV,)),
            pltpu.SemaphoreType.DMA((N_DEV,)),
        ],
        compiler_params=pltpu.CompilerParams(collective_id=0),
    )(x)
